# baseline (speedup 1.0000x reference)
import jax
import jax.numpy as jnp
from jax.experimental import pallas as pl
from jax.experimental.pallas import tpu as pltpu
from jax.experimental.pallas import tpu_sc as plsc


def _mlp_table_kernel(pe_ref, w1_ref, b1_ref, w2_ref, b2_ref, out_ref):
    x = pe_ref[...]
    h = jnp.dot(x, w1_ref[...], preferred_element_type=jnp.float32)
    h = h + b1_ref[...]
    h = h * jax.nn.sigmoid(h)
    o = jnp.dot(h, w2_ref[...], preferred_element_type=jnp.float32)
    out_ref[...] = o + b2_ref[...]


def _mlp_table(pe, W1, b1, W2, b2):
    max_len, d = pe.shape
    t = W2.shape[1]
    blk = 2000
    grid = (pl.cdiv(max_len, blk),)
    return pl.pallas_call(
        _mlp_table_kernel,
        grid=grid,
        in_specs=[
            pl.BlockSpec((blk, d), lambda i: (i, 0)),
            pl.BlockSpec((d, t), lambda i: (0, 0)),
            pl.BlockSpec((1, t), lambda i: (0, 0)),
            pl.BlockSpec((t, t), lambda i: (0, 0)),
            pl.BlockSpec((1, t), lambda i: (0, 0)),
        ],
        out_specs=pl.BlockSpec((blk, t), lambda i: (i, 0)),
        out_shape=jax.ShapeDtypeStruct((max_len, t), jnp.float32),
    )(pe, W1, b1.reshape(1, t), W2, b2.reshape(1, t))


def _sc_gather(table, indices):
    n = indices.shape[0]
    d = table.shape[1]
    n_units = 32
    per_unit = n // n_units
    window = 512
    n_chunks = per_unit // window
    idx2d = indices.reshape(1, n)
    mesh = plsc.VectorSubcoreMesh(core_axis_name="core",
                                  subcore_axis_name="subcore")

    @pl.kernel(
        out_type=jax.ShapeDtypeStruct((n, d), table.dtype),
        mesh=mesh,
        scratch_types=[
            pltpu.VMEM((per_unit,), jnp.int32),
            pltpu.VMEM((n_chunks, window, d), table.dtype),
            pltpu.SemaphoreType.DMA,
            pltpu.SemaphoreType.DMA,
            pltpu.SemaphoreType.DMA,
        ],
    )
    def gather_kernel(tbl_hbm, idx_hbm, out_hbm, idx_vmem, buf, sem_i,
                      sem_g, sem_o):
        unit = jax.lax.axis_index("core") * 16 + jax.lax.axis_index("subcore")
        base = unit * per_unit
        pltpu.async_copy(idx_hbm.at[0, pl.ds(base, per_unit)], idx_vmem,
                         sem_i).wait()

        gathers = [
            pltpu.async_copy(
                tbl_hbm.at[idx_vmem.at[pl.ds(c * window, window)]],
                buf.at[c], sem_g)
            for c in range(n_chunks)
        ]
        writes = []
        for c in range(n_chunks):
            gathers[c].wait()
            writes.append(pltpu.async_copy(
                buf.at[c],
                out_hbm.at[pl.ds(base + c * window, window)], sem_o))
        for w in writes:
            w.wait()

    return gather_kernel(table, idx2d)


def kernel(timesteps, pe, W1, b1, W2, b2):
    table = _mlp_table(pe, W1, b1, W2, b2)
    return _sc_gather(table, timesteps)


# speedup vs baseline: 1.0456x; 1.0456x over previous
import jax
import jax.numpy as jnp
from jax.experimental import pallas as pl
from jax.experimental.pallas import tpu as pltpu
from jax.experimental.pallas import tpu_sc as plsc


def _mlp_table_kernel(pe_ref, w1_ref, b1_ref, w2_ref, b2_ref, out_ref):
    x = pe_ref[...]
    h = jnp.dot(x, w1_ref[...], preferred_element_type=jnp.float32)
    h = h + b1_ref[...]
    h = h * jax.nn.sigmoid(h)
    o = jnp.dot(h, w2_ref[...], preferred_element_type=jnp.float32)
    out_ref[...] = o + b2_ref[...]


def _mlp_table(pe, W1, b1, W2, b2):
    max_len, d = pe.shape
    t = W2.shape[1]
    blk = 10000
    grid = (pl.cdiv(max_len, blk),)
    return pl.pallas_call(
        _mlp_table_kernel,
        grid=grid,
        in_specs=[
            pl.BlockSpec((blk, d), lambda i: (i, 0)),
            pl.BlockSpec((d, t), lambda i: (0, 0)),
            pl.BlockSpec((1, t), lambda i: (0, 0)),
            pl.BlockSpec((t, t), lambda i: (0, 0)),
            pl.BlockSpec((1, t), lambda i: (0, 0)),
        ],
        out_specs=pl.BlockSpec((blk, t), lambda i: (i, 0)),
        out_shape=jax.ShapeDtypeStruct((max_len, t), jnp.float32),
    )(pe, W1, b1.reshape(1, t), W2, b2.reshape(1, t))


def _sc_gather(table, indices):
    n = indices.shape[0]
    d = table.shape[1]
    n_units = 32
    per_unit = n // n_units
    window = 512
    n_chunks = per_unit // window
    idx2d = indices.reshape(1, n)
    mesh = plsc.VectorSubcoreMesh(core_axis_name="core",
                                  subcore_axis_name="subcore")

    @pl.kernel(
        out_type=jax.ShapeDtypeStruct((n, d), table.dtype),
        mesh=mesh,
        scratch_types=[
            pltpu.VMEM((per_unit,), jnp.int32),
            pltpu.VMEM((n_chunks, window, d), table.dtype),
            pltpu.SemaphoreType.DMA,
            pltpu.SemaphoreType.DMA,
            pltpu.SemaphoreType.DMA,
        ],
    )
    def gather_kernel(tbl_hbm, idx_hbm, out_hbm, idx_vmem, buf, sem_i,
                      sem_g, sem_o):
        unit = jax.lax.axis_index("core") * 16 + jax.lax.axis_index("subcore")
        base = unit * per_unit
        pltpu.async_copy(idx_hbm.at[0, pl.ds(base, per_unit)], idx_vmem,
                         sem_i).wait()

        gathers = [
            pltpu.async_copy(
                tbl_hbm.at[idx_vmem.at[pl.ds(c * window, window)]],
                buf.at[c], sem_g)
            for c in range(n_chunks)
        ]
        writes = []
        for c in range(n_chunks):
            gathers[c].wait()
            writes.append(pltpu.async_copy(
                buf.at[c],
                out_hbm.at[pl.ds(base + c * window, window)], sem_o))
        for w in writes:
            w.wait()

    return gather_kernel(table, idx2d)


def kernel(timesteps, pe, W1, b1, W2, b2):
    table = _mlp_table(pe, W1, b1, W2, b2)
    return _sc_gather(table, timesteps)
